# baseline (speedup 1.0000x reference)
import functools

import jax
import jax.numpy as jnp
from jax import lax
from jax.experimental import pallas as pl
from jax.experimental.pallas import tpu as pltpu
from jax.experimental.pallas import tpu_sc as plsc

BATCH = 16384
HIDDEN = 64
WIDE = 2 * HIDDEN

NUM_CORES = 2
NUM_SUBCORES = 16
NUM_WORKERS = NUM_CORES * NUM_SUBCORES
B_PER_W = BATCH // NUM_WORKERS
CHUNK = 64
N_CHUNKS = B_PER_W // CHUNK

_SC_MESH = plsc.VectorSubcoreMesh(core_axis_name="c", subcore_axis_name="s")


_PRJ_BLOCK = 8192


def _project_body(tt, w, o):
    p = jax.lax.dot_general(tt[...].astype(jnp.bfloat16),
                            w[...].astype(jnp.bfloat16),
                            (((0,), (0,)), ((), ())),
                            preferred_element_type=jnp.float32)
    o[...] = pltpu.bitcast(p.astype(jnp.bfloat16), jnp.float32)


def _project(tt, w, n_rows):
    nb = pl.cdiv(n_rows, _PRJ_BLOCK)
    return pl.pallas_call(
        _project_body,
        grid=(nb,),
        in_specs=[
            pl.BlockSpec((HIDDEN, _PRJ_BLOCK), lambda i: (0, i)),
            pl.BlockSpec((HIDDEN, WIDE), lambda i: (0, 0)),
        ],
        out_specs=pl.BlockSpec((_PRJ_BLOCK // 2, WIDE), lambda i: (i, 0)),
        out_shape=jax.ShapeDtypeStruct((n_rows // 2, WIDE), jnp.float32),
        compiler_params=pltpu.CompilerParams(
            dimension_semantics=("parallel",)),
    )(tt, w)


@functools.partial(
    pl.kernel,
    mesh=_SC_MESH,
    out_type=[
        jax.ShapeDtypeStruct((BATCH, WIDE), jnp.float32),
        jax.ShapeDtypeStruct((BATCH, WIDE), jnp.float32),
    ],
    scratch_types=[
        pltpu.VMEM((N_CHUNKS, CHUNK), jnp.int32),
        pltpu.VMEM((N_CHUNKS, CHUNK), jnp.int32),
        pltpu.VMEM((CHUNK, WIDE), jnp.float32),
        pltpu.VMEM((CHUNK, WIDE), jnp.float32),
        pltpu.VMEM((CHUNK, WIDE), jnp.float32),
        pltpu.VMEM((CHUNK, WIDE), jnp.float32),
        pltpu.SemaphoreType.DMA,
        pltpu.SemaphoreType.DMA,
        pltpu.SemaphoreType.DMA,
        pltpu.SemaphoreType.DMA,
    ],
)
def _sc_gather(u_idx_hbm, i_idx_hbm, pu_hbm, pi_hbm, uo_hbm, io_hbm,
               uidx_v, iidx_v, urows0, urows1, irows0, irows1,
               sem_u0, sem_u1, sem_i0, sem_i1):
    wid = lax.axis_index("s") * NUM_CORES + lax.axis_index("c")
    base = wid * B_PER_W
    pltpu.sync_copy(u_idx_hbm.at[pl.ds(wid * N_CHUNKS, N_CHUNKS)], uidx_v)
    pltpu.sync_copy(i_idx_hbm.at[pl.ds(wid * N_CHUNKS, N_CHUNKS)], iidx_v)

    ubufs = (urows0, urows1)
    ibufs = (irows0, irows1)
    usems = (sem_u0, sem_u1)
    isems = (sem_i0, sem_i1)
    gathers = [None, None]
    for j in range(N_CHUNKS):
        b = j % 2
        gathers[b] = (
            pltpu.async_copy(pu_hbm.at[uidx_v.at[j]], ubufs[b], usems[b]),
            pltpu.async_copy(pi_hbm.at[iidx_v.at[j]], ibufs[b], isems[b]),
        )
        if j > 0:
            p = (j - 1) % 2
            gu, gi = gathers[p]
            gu.wait()
            gi.wait()
            off = base + (j - 1) * CHUNK
            pltpu.sync_copy(ubufs[p], uo_hbm.at[pl.ds(off, CHUNK)])
            pltpu.sync_copy(ibufs[p], io_hbm.at[pl.ds(off, CHUNK)])
    lb = (N_CHUNKS - 1) % 2
    gu, gi = gathers[lb]
    gu.wait()
    gi.wait()
    off = base + (N_CHUNKS - 1) * CHUNK
    pltpu.sync_copy(ubufs[lb], uo_hbm.at[pl.ds(off, CHUNK)])
    pltpu.sync_copy(ibufs[lb], io_hbm.at[pl.ds(off, CHUNK)])


_MLP_BLOCK = 2048


def _unpack_select(packed, parity):
    g = pltpu.bitcast(packed, jnp.uint32)
    lo = pltpu.bitcast(g << jnp.uint32(16), jnp.float32)
    hi = pltpu.bitcast(g & jnp.uint32(0xFFFF0000), jnp.float32)
    return jnp.where(parity > 0, hi, lo)


def _tail_body(pu, pi, su, si, b1, w2, b2, w3, b3, o):
    eu = _unpack_select(pu[...][:, :HIDDEN], su[...])
    ei = _unpack_select(pi[...][:, HIDDEN:], si[...])
    h = jnp.maximum(eu + ei + b1[...], 0.0)
    h = jnp.dot(h, w2[...], preferred_element_type=jnp.float32) + b2[...]
    h = jnp.maximum(h, 0.0)
    z = jnp.dot(h, w3[...], preferred_element_type=jnp.float32) + b3[...]
    o[...] = jax.nn.sigmoid(z)


def _tail(pu, pi, su, si, b1, w2, b2, w3, b3):
    nb = BATCH // _MLP_BLOCK
    const = lambda *_: (0, 0)
    return pl.pallas_call(
        _tail_body,
        grid=(nb,),
        in_specs=[
            pl.BlockSpec((_MLP_BLOCK, WIDE), lambda i: (i, 0)),
            pl.BlockSpec((_MLP_BLOCK, WIDE), lambda i: (i, 0)),
            pl.BlockSpec((_MLP_BLOCK, 1), lambda i: (i, 0)),
            pl.BlockSpec((_MLP_BLOCK, 1), lambda i: (i, 0)),
            pl.BlockSpec((1, HIDDEN), const),
            pl.BlockSpec((HIDDEN, HIDDEN // 2), const),
            pl.BlockSpec((1, HIDDEN // 2), const),
            pl.BlockSpec((HIDDEN // 2, 1), const),
            pl.BlockSpec((1, 1), const),
        ],
        out_specs=pl.BlockSpec((_MLP_BLOCK, 1), lambda i: (i, 0)),
        out_shape=jax.ShapeDtypeStruct((BATCH, 1), jnp.float32),
        compiler_params=pltpu.CompilerParams(
            dimension_semantics=("parallel",)),
    )(pu, pi, su, si, b1, w2, b2, w3, b3)


def kernel(user, item, user_table, item_table, W1, b1, W2, b2, W3, b3):
    user = user.astype(jnp.int32)
    item = item.astype(jnp.int32)
    w1u_pad = jnp.pad(W1[:HIDDEN], ((0, 0), (0, HIDDEN)))
    w1v_pad = jnp.pad(W1[HIDDEN:], ((0, 0), (HIDDEN, 0)))
    pu_w = _project(user_table.T, w1u_pad, user_table.shape[0])
    pi_w = _project(item_table.T, w1v_pad, item_table.shape[0])
    u2d = (user // 2).reshape(BATCH // CHUNK, CHUNK)
    i2d = (item // 2).reshape(BATCH // CHUNK, CHUNK)
    gu, gi = _sc_gather(u2d, i2d, pu_w, pi_w)
    su = (user % 2).astype(jnp.float32).reshape(BATCH, 1)
    si = (item % 2).astype(jnp.float32).reshape(BATCH, 1)
    return _tail(gu, gi, su, si,
                 b1.reshape(1, HIDDEN), W2, b2.reshape(1, HIDDEN // 2),
                 W3, b3.reshape(1, 1))


# speedup vs baseline: 1.1551x; 1.1551x over previous
import functools

import jax
import jax.numpy as jnp
from jax import lax
from jax.experimental import pallas as pl
from jax.experimental.pallas import tpu as pltpu
from jax.experimental.pallas import tpu_sc as plsc

BATCH = 16384
HIDDEN = 64
WIDE = 2 * HIDDEN

NUM_CORES = 2
NUM_SUBCORES = 16
NUM_WORKERS = NUM_CORES * NUM_SUBCORES
B_PER_W = BATCH // NUM_WORKERS
CHUNK = 64
N_CHUNKS = B_PER_W // CHUNK

_SC_MESH = plsc.VectorSubcoreMesh(core_axis_name="c", subcore_axis_name="s")


_PRJ_BLOCK = 16384


def _project_body(tt, w, o):
    p = jax.lax.dot_general(tt[...].astype(jnp.bfloat16),
                            w[...].astype(jnp.bfloat16),
                            (((0,), (0,)), ((), ())),
                            preferred_element_type=jnp.float32)
    o[...] = pltpu.bitcast(p.astype(jnp.bfloat16), jnp.float32)


def _project(tt, w, n_rows):
    nb = pl.cdiv(n_rows, _PRJ_BLOCK)
    return pl.pallas_call(
        _project_body,
        grid=(nb,),
        in_specs=[
            pl.BlockSpec((HIDDEN, _PRJ_BLOCK), lambda i: (0, i)),
            pl.BlockSpec((HIDDEN, WIDE), lambda i: (0, 0)),
        ],
        out_specs=pl.BlockSpec((_PRJ_BLOCK // 2, WIDE), lambda i: (i, 0)),
        out_shape=jax.ShapeDtypeStruct((n_rows // 2, WIDE), jnp.float32),
        compiler_params=pltpu.CompilerParams(
            dimension_semantics=("parallel",)),
    )(tt, w)


@functools.partial(
    pl.kernel,
    mesh=_SC_MESH,
    out_type=[
        jax.ShapeDtypeStruct((BATCH, WIDE), jnp.float32),
        jax.ShapeDtypeStruct((BATCH, WIDE), jnp.float32),
    ],
    scratch_types=[
        pltpu.VMEM((N_CHUNKS, CHUNK), jnp.int32),
        pltpu.VMEM((N_CHUNKS, CHUNK), jnp.int32),
        pltpu.VMEM((CHUNK, WIDE), jnp.float32),
        pltpu.VMEM((CHUNK, WIDE), jnp.float32),
        pltpu.VMEM((CHUNK, WIDE), jnp.float32),
        pltpu.VMEM((CHUNK, WIDE), jnp.float32),
        pltpu.SemaphoreType.DMA,
        pltpu.SemaphoreType.DMA,
        pltpu.SemaphoreType.DMA,
        pltpu.SemaphoreType.DMA,
    ],
)
def _sc_gather(u_idx_hbm, i_idx_hbm, pu_hbm, pi_hbm, uo_hbm, io_hbm,
               uidx_v, iidx_v, urows0, urows1, irows0, irows1,
               sem_u0, sem_u1, sem_i0, sem_i1):
    wid = lax.axis_index("s") * NUM_CORES + lax.axis_index("c")
    base = wid * B_PER_W
    pltpu.sync_copy(u_idx_hbm.at[pl.ds(wid * N_CHUNKS, N_CHUNKS)], uidx_v)
    pltpu.sync_copy(i_idx_hbm.at[pl.ds(wid * N_CHUNKS, N_CHUNKS)], iidx_v)

    ubufs = (urows0, urows1)
    ibufs = (irows0, irows1)
    usems = (sem_u0, sem_u1)
    isems = (sem_i0, sem_i1)
    gathers = [None, None]
    for j in range(N_CHUNKS):
        b = j % 2
        gathers[b] = (
            pltpu.async_copy(pu_hbm.at[uidx_v.at[j]], ubufs[b], usems[b]),
            pltpu.async_copy(pi_hbm.at[iidx_v.at[j]], ibufs[b], isems[b]),
        )
        if j > 0:
            p = (j - 1) % 2
            gu, gi = gathers[p]
            gu.wait()
            gi.wait()
            off = base + (j - 1) * CHUNK
            pltpu.sync_copy(ubufs[p], uo_hbm.at[pl.ds(off, CHUNK)])
            pltpu.sync_copy(ibufs[p], io_hbm.at[pl.ds(off, CHUNK)])
    lb = (N_CHUNKS - 1) % 2
    gu, gi = gathers[lb]
    gu.wait()
    gi.wait()
    off = base + (N_CHUNKS - 1) * CHUNK
    pltpu.sync_copy(ubufs[lb], uo_hbm.at[pl.ds(off, CHUNK)])
    pltpu.sync_copy(ibufs[lb], io_hbm.at[pl.ds(off, CHUNK)])


_MLP_BLOCK = 2048


def _unpack_select(packed, parity):
    g = pltpu.bitcast(packed, jnp.uint32)
    lo = pltpu.bitcast(g << jnp.uint32(16), jnp.float32)
    hi = pltpu.bitcast(g & jnp.uint32(0xFFFF0000), jnp.float32)
    return jnp.where(parity > 0, hi, lo)


def _tail_body(pu, pi, su, si, b1, w2, b2, w3, b3, o):
    eu = _unpack_select(pu[...][:, :HIDDEN], su[...])
    ei = _unpack_select(pi[...][:, HIDDEN:], si[...])
    h = jnp.maximum(eu + ei + b1[...], 0.0)
    h = jnp.dot(h, w2[...], preferred_element_type=jnp.float32) + b2[...]
    h = jnp.maximum(h, 0.0)
    z = jnp.dot(h, w3[...], preferred_element_type=jnp.float32) + b3[...]
    o[...] = jax.nn.sigmoid(z)


def _tail(pu, pi, su, si, b1, w2, b2, w3, b3):
    nb = BATCH // _MLP_BLOCK
    const = lambda *_: (0, 0)
    return pl.pallas_call(
        _tail_body,
        grid=(nb,),
        in_specs=[
            pl.BlockSpec((_MLP_BLOCK, WIDE), lambda i: (i, 0)),
            pl.BlockSpec((_MLP_BLOCK, WIDE), lambda i: (i, 0)),
            pl.BlockSpec((_MLP_BLOCK, 1), lambda i: (i, 0)),
            pl.BlockSpec((_MLP_BLOCK, 1), lambda i: (i, 0)),
            pl.BlockSpec((1, HIDDEN), const),
            pl.BlockSpec((HIDDEN, HIDDEN // 2), const),
            pl.BlockSpec((1, HIDDEN // 2), const),
            pl.BlockSpec((HIDDEN // 2, 1), const),
            pl.BlockSpec((1, 1), const),
        ],
        out_specs=pl.BlockSpec((_MLP_BLOCK, 1), lambda i: (i, 0)),
        out_shape=jax.ShapeDtypeStruct((BATCH, 1), jnp.float32),
        compiler_params=pltpu.CompilerParams(
            dimension_semantics=("parallel",)),
    )(pu, pi, su, si, b1, w2, b2, w3, b3)


def kernel(user, item, user_table, item_table, W1, b1, W2, b2, W3, b3):
    user = user.astype(jnp.int32)
    item = item.astype(jnp.int32)
    w1u_pad = jnp.pad(W1[:HIDDEN], ((0, 0), (0, HIDDEN)))
    w1v_pad = jnp.pad(W1[HIDDEN:], ((0, 0), (HIDDEN, 0)))
    pu_w = _project(user_table.T, w1u_pad, user_table.shape[0])
    pi_w = _project(item_table.T, w1v_pad, item_table.shape[0])
    u2d = (user // 2).reshape(BATCH // CHUNK, CHUNK)
    i2d = (item // 2).reshape(BATCH // CHUNK, CHUNK)
    gu, gi = _sc_gather(u2d, i2d, pu_w, pi_w)
    su = (user % 2).astype(jnp.float32).reshape(BATCH, 1)
    si = (item % 2).astype(jnp.float32).reshape(BATCH, 1)
    return _tail(gu, gi, su, si,
                 b1.reshape(1, HIDDEN), W2, b2.reshape(1, HIDDEN // 2),
                 W3, b3.reshape(1, 1))


# speedup vs baseline: 1.1887x; 1.0291x over previous
import functools

import jax
import jax.numpy as jnp
from jax import lax
from jax.experimental import pallas as pl
from jax.experimental.pallas import tpu as pltpu
from jax.experimental.pallas import tpu_sc as plsc

BATCH = 16384
HIDDEN = 64
WIDE = 2 * HIDDEN

NUM_CORES = 2
NUM_SUBCORES = 16
NUM_WORKERS = NUM_CORES * NUM_SUBCORES
B_PER_W = BATCH // NUM_WORKERS
CHUNK = 64
N_CHUNKS = B_PER_W // CHUNK

_SC_MESH = plsc.VectorSubcoreMesh(core_axis_name="c", subcore_axis_name="s")


_PRJ_BLOCK = 32768


def _project_body(tt, w, o):
    p = jax.lax.dot_general(tt[...].astype(jnp.bfloat16),
                            w[...].astype(jnp.bfloat16),
                            (((0,), (0,)), ((), ())),
                            preferred_element_type=jnp.float32)
    o[...] = pltpu.bitcast(p.astype(jnp.bfloat16), jnp.float32)


def _project(tt, w, n_rows):
    nb = pl.cdiv(n_rows, _PRJ_BLOCK)
    return pl.pallas_call(
        _project_body,
        grid=(nb,),
        in_specs=[
            pl.BlockSpec((HIDDEN, _PRJ_BLOCK), lambda i: (0, i)),
            pl.BlockSpec((HIDDEN, WIDE), lambda i: (0, 0)),
        ],
        out_specs=pl.BlockSpec((_PRJ_BLOCK // 2, WIDE), lambda i: (i, 0)),
        out_shape=jax.ShapeDtypeStruct((n_rows // 2, WIDE), jnp.float32),
        compiler_params=pltpu.CompilerParams(
            dimension_semantics=("parallel",)),
    )(tt, w)


@functools.partial(
    pl.kernel,
    mesh=_SC_MESH,
    out_type=[
        jax.ShapeDtypeStruct((BATCH, WIDE), jnp.float32),
        jax.ShapeDtypeStruct((BATCH, WIDE), jnp.float32),
    ],
    scratch_types=[
        pltpu.VMEM((N_CHUNKS, CHUNK), jnp.int32),
        pltpu.VMEM((N_CHUNKS, CHUNK), jnp.int32),
        pltpu.VMEM((CHUNK, WIDE), jnp.float32),
        pltpu.VMEM((CHUNK, WIDE), jnp.float32),
        pltpu.VMEM((CHUNK, WIDE), jnp.float32),
        pltpu.VMEM((CHUNK, WIDE), jnp.float32),
        pltpu.SemaphoreType.DMA,
        pltpu.SemaphoreType.DMA,
        pltpu.SemaphoreType.DMA,
        pltpu.SemaphoreType.DMA,
    ],
)
def _sc_gather(u_idx_hbm, i_idx_hbm, pu_hbm, pi_hbm, uo_hbm, io_hbm,
               uidx_v, iidx_v, urows0, urows1, irows0, irows1,
               sem_u0, sem_u1, sem_i0, sem_i1):
    wid = lax.axis_index("s") * NUM_CORES + lax.axis_index("c")
    base = wid * B_PER_W
    pltpu.sync_copy(u_idx_hbm.at[pl.ds(wid * N_CHUNKS, N_CHUNKS)], uidx_v)
    pltpu.sync_copy(i_idx_hbm.at[pl.ds(wid * N_CHUNKS, N_CHUNKS)], iidx_v)

    ubufs = (urows0, urows1)
    ibufs = (irows0, irows1)
    usems = (sem_u0, sem_u1)
    isems = (sem_i0, sem_i1)
    gathers = [None, None]
    for j in range(N_CHUNKS):
        b = j % 2
        gathers[b] = (
            pltpu.async_copy(pu_hbm.at[uidx_v.at[j]], ubufs[b], usems[b]),
            pltpu.async_copy(pi_hbm.at[iidx_v.at[j]], ibufs[b], isems[b]),
        )
        if j > 0:
            p = (j - 1) % 2
            gu, gi = gathers[p]
            gu.wait()
            gi.wait()
            off = base + (j - 1) * CHUNK
            pltpu.sync_copy(ubufs[p], uo_hbm.at[pl.ds(off, CHUNK)])
            pltpu.sync_copy(ibufs[p], io_hbm.at[pl.ds(off, CHUNK)])
    lb = (N_CHUNKS - 1) % 2
    gu, gi = gathers[lb]
    gu.wait()
    gi.wait()
    off = base + (N_CHUNKS - 1) * CHUNK
    pltpu.sync_copy(ubufs[lb], uo_hbm.at[pl.ds(off, CHUNK)])
    pltpu.sync_copy(ibufs[lb], io_hbm.at[pl.ds(off, CHUNK)])


_MLP_BLOCK = 2048


def _unpack_select(packed, parity):
    g = pltpu.bitcast(packed, jnp.uint32)
    lo = pltpu.bitcast(g << jnp.uint32(16), jnp.float32)
    hi = pltpu.bitcast(g & jnp.uint32(0xFFFF0000), jnp.float32)
    return jnp.where(parity > 0, hi, lo)


def _tail_body(pu, pi, su, si, b1, w2, b2, w3, b3, o):
    eu = _unpack_select(pu[...][:, :HIDDEN], su[...])
    ei = _unpack_select(pi[...][:, HIDDEN:], si[...])
    h = jnp.maximum(eu + ei + b1[...], 0.0)
    h = jnp.dot(h, w2[...], preferred_element_type=jnp.float32) + b2[...]
    h = jnp.maximum(h, 0.0)
    z = jnp.dot(h, w3[...], preferred_element_type=jnp.float32) + b3[...]
    o[...] = jax.nn.sigmoid(z)


def _tail(pu, pi, su, si, b1, w2, b2, w3, b3):
    nb = BATCH // _MLP_BLOCK
    const = lambda *_: (0, 0)
    return pl.pallas_call(
        _tail_body,
        grid=(nb,),
        in_specs=[
            pl.BlockSpec((_MLP_BLOCK, WIDE), lambda i: (i, 0)),
            pl.BlockSpec((_MLP_BLOCK, WIDE), lambda i: (i, 0)),
            pl.BlockSpec((_MLP_BLOCK, 1), lambda i: (i, 0)),
            pl.BlockSpec((_MLP_BLOCK, 1), lambda i: (i, 0)),
            pl.BlockSpec((1, HIDDEN), const),
            pl.BlockSpec((HIDDEN, HIDDEN // 2), const),
            pl.BlockSpec((1, HIDDEN // 2), const),
            pl.BlockSpec((HIDDEN // 2, 1), const),
            pl.BlockSpec((1, 1), const),
        ],
        out_specs=pl.BlockSpec((_MLP_BLOCK, 1), lambda i: (i, 0)),
        out_shape=jax.ShapeDtypeStruct((BATCH, 1), jnp.float32),
        compiler_params=pltpu.CompilerParams(
            dimension_semantics=("parallel",)),
    )(pu, pi, su, si, b1, w2, b2, w3, b3)


def kernel(user, item, user_table, item_table, W1, b1, W2, b2, W3, b3):
    user = user.astype(jnp.int32)
    item = item.astype(jnp.int32)
    w1u_pad = jnp.pad(W1[:HIDDEN], ((0, 0), (0, HIDDEN)))
    w1v_pad = jnp.pad(W1[HIDDEN:], ((0, 0), (HIDDEN, 0)))
    pu_w = _project(user_table.T, w1u_pad, user_table.shape[0])
    pi_w = _project(item_table.T, w1v_pad, item_table.shape[0])
    u2d = (user // 2).reshape(BATCH // CHUNK, CHUNK)
    i2d = (item // 2).reshape(BATCH // CHUNK, CHUNK)
    gu, gi = _sc_gather(u2d, i2d, pu_w, pi_w)
    su = (user % 2).astype(jnp.float32).reshape(BATCH, 1)
    si = (item % 2).astype(jnp.float32).reshape(BATCH, 1)
    return _tail(gu, gi, su, si,
                 b1.reshape(1, HIDDEN), W2, b2.reshape(1, HIDDEN // 2),
                 W3, b3.reshape(1, 1))
